# Initial kernel scaffold; baseline (speedup 1.0000x reference)
import functools

import jax
import jax.numpy as jnp
from jax import lax
from jax.experimental import pallas as pl
from jax.experimental.pallas import tpu as pltpu

N = 4096
D = 128
KNN = 8
RT = 256
NT = N // RT
NEG = -3.0e38


def _agg_kernel(feats_ref, wq_ref, bq_ref, wkv_ref, bkv_ref, out_ref,
                q_ref, k_ref, v_ref):
    f = feats_ref[0]
    q = jnp.dot(f, wq_ref[...], preferred_element_type=jnp.float32) + bq_ref[...]
    kv = jnp.dot(f, wkv_ref[...], preferred_element_type=jnp.float32) + bkv_ref[...]
    km = kv[:, :D]
    vm = kv[:, D:]
    q_ref[...] = q
    k_ref[...] = km
    v_ref[...] = vm

    kn2 = lax.dot_general(jnp.ones((1, D), jnp.float32), km * km,
                          (((1,), (1,)), ((), ())),
                          preferred_element_type=jnp.float32)
    inv_kn = 1.0 / jnp.maximum(jnp.sqrt(kn2), 1e-12)

    ones_row = jnp.ones((1, RT), jnp.float32)
    inv_sqrt_d = 1.0 / (D ** 0.5)

    def tile_body(i, w):
        qt = q_ref[pl.ds(i * RT, RT), :]
        s = lax.dot_general(qt, k_ref[...], (((1,), (1,)), ((), ())),
                            preferred_element_type=jnp.float32)
        colj = lax.broadcasted_iota(jnp.int32, (RT, N), 1)
        rowi = lax.broadcasted_iota(jnp.int32, (RT, N), 0) + i * RT
        r = jnp.where(colj == rowi, NEG, s * inv_kn)

        logits = []
        idxs = []
        for _ in range(KNN):
            m = jnp.max(r, axis=1, keepdims=True)
            amax = jnp.min(jnp.where(r == m, colj, N), axis=1, keepdims=True)
            onehot = colj == amax
            lk = jnp.sum(jnp.where(onehot, s, 0.0), axis=1, keepdims=True)
            logits.append(lk)
            idxs.append(amax)
            r = jnp.where(onehot, NEG, r)

        lmat = jnp.concatenate(logits, axis=1) * inv_sqrt_d
        lmax = jnp.max(lmat, axis=1, keepdims=True)
        u = jnp.exp(lmat - lmax)
        attn = u / jnp.sum(u, axis=1, keepdims=True)

        en = jnp.zeros((RT, N), jnp.float32)
        for kk in range(KNN):
            en = en + jnp.where(colj == idxs[kk], attn[:, kk:kk + 1], 0.0)
        winc = lax.dot_general(ones_row, en, (((1,), (0,)), ((), ())),
                               preferred_element_type=jnp.float32)
        return w + winc

    w = lax.fori_loop(0, NT, tile_body, jnp.zeros((1, N), jnp.float32))
    z = lax.dot_general(w, v_ref[...], (((1,), (0,)), ((), ())),
                        preferred_element_type=jnp.float32)
    out_ref[...] = z * (1.0 / N)


@jax.jit
def kernel(feats, Wq, bq, Wkv, bkv):
    if feats.ndim == 2:
        feats = feats[None]
    B = feats.shape[0]
    return pl.pallas_call(
        _agg_kernel,
        grid=(B,),
        in_specs=[
            pl.BlockSpec((1, N, D), lambda b: (b, 0, 0)),
            pl.BlockSpec((D, D), lambda b: (0, 0)),
            pl.BlockSpec((D,), lambda b: (0,)),
            pl.BlockSpec((D, 2 * D), lambda b: (0, 0)),
            pl.BlockSpec((2 * D,), lambda b: (0,)),
        ],
        out_specs=pl.BlockSpec((1, D), lambda b: (b, 0)),
        out_shape=jax.ShapeDtypeStruct((B, D), jnp.float32),
        scratch_shapes=[
            pltpu.VMEM((N, D), jnp.float32),
            pltpu.VMEM((N, D), jnp.float32),
            pltpu.VMEM((N, D), jnp.float32),
        ],
    )(feats, Wq, bq, Wkv, bkv)


# speedup vs baseline: 16.5042x; 16.5042x over previous
import jax
import jax.numpy as jnp
from jax import lax
from jax.experimental import pallas as pl
from jax.experimental.pallas import tpu as pltpu

N = 4096
D = 128
KNN = 8
RT = 256
NT = N // RT
NEG = -3.0e38
EPS = 1e-12


def _agg_kernel(feats_ref, wq_ref, bq_ref, wkv_ref, bkv_ref, out_ref,
                nq_ref, nk_ref, v_ref, qn_ref):
    f = feats_ref[0].astype(jnp.bfloat16)
    q = jnp.dot(f, wq_ref[...].astype(jnp.bfloat16),
                preferred_element_type=jnp.float32) + bq_ref[...]
    kv = jnp.dot(f, wkv_ref[...].astype(jnp.bfloat16),
                 preferred_element_type=jnp.float32) + bkv_ref[...]
    km = kv[:, :D]
    v_ref[...] = kv[:, D:]

    qn = jnp.maximum(jnp.sqrt(jnp.sum(q * q, axis=1, keepdims=True)), EPS)
    kn = jnp.maximum(jnp.sqrt(jnp.sum(km * km, axis=1, keepdims=True)), EPS)
    nq_ref[...] = (q / qn).astype(jnp.bfloat16)
    nk_ref[...] = (km / kn).astype(jnp.bfloat16)
    qn_ref[...] = qn

    kn2_row = lax.dot_general(jnp.ones((1, D), jnp.float32), km * km,
                              (((1,), (1,)), ((), ())),
                              preferred_element_type=jnp.float32,
                              precision=lax.Precision.HIGHEST)
    kn_row = jnp.maximum(jnp.sqrt(kn2_row), EPS)

    ones_row = jnp.ones((1, RT), jnp.float32)
    inv_sqrt_d = 1.0 / (D ** 0.5)

    def tile_body(i, w):
        nqt = nq_ref[pl.ds(i * RT, RT), :]
        sim = lax.dot_general(nqt, nk_ref[...], (((1,), (1,)), ((), ())),
                              preferred_element_type=jnp.float32)
        colj = lax.broadcasted_iota(jnp.int32, (RT, N), 1)
        rowi = lax.broadcasted_iota(jnp.int32, (RT, N), 0) + i * RT
        r = jnp.where(colj == rowi, NEG, sim)
        g = sim * ((qn_ref[pl.ds(i * RT, RT), :] * inv_sqrt_d) * kn_row)

        logits = []
        idxs = []
        for _ in range(KNN):
            m = jnp.max(r, axis=1, keepdims=True)
            amax = jnp.min(jnp.where(r == m, colj, N), axis=1, keepdims=True)
            onehot = colj == amax
            lk = jnp.sum(jnp.where(onehot, g, 0.0), axis=1, keepdims=True)
            logits.append(lk)
            idxs.append(amax)
            r = jnp.where(onehot, NEG, r)

        lmat = jnp.concatenate(logits, axis=1)
        lmax = jnp.max(lmat, axis=1, keepdims=True)
        u = jnp.exp(lmat - lmax)
        attn = u / jnp.sum(u, axis=1, keepdims=True)

        en = jnp.zeros((RT, N), jnp.float32)
        for kk in range(KNN):
            en = en + jnp.where(colj == idxs[kk], attn[:, kk:kk + 1], 0.0)
        winc = lax.dot_general(ones_row, en, (((1,), (0,)), ((), ())),
                               preferred_element_type=jnp.float32,
                               precision=lax.Precision.HIGHEST)
        return w + winc

    w = lax.fori_loop(0, NT, tile_body, jnp.zeros((1, N), jnp.float32))
    z = lax.dot_general(w, v_ref[...], (((1,), (0,)), ((), ())),
                        preferred_element_type=jnp.float32,
                        precision=lax.Precision.HIGHEST)
    b = pl.program_id(0)
    out_ref[pl.ds(b, 1), :] = z * (1.0 / N)


@jax.jit
def kernel(feats, Wq, bq, Wkv, bkv):
    if feats.ndim == 2:
        feats = feats[None]
    B = feats.shape[0]
    return pl.pallas_call(
        _agg_kernel,
        grid=(B,),
        in_specs=[
            pl.BlockSpec((1, N, D), lambda b: (b, 0, 0)),
            pl.BlockSpec((D, D), lambda b: (0, 0)),
            pl.BlockSpec((D,), lambda b: (0,)),
            pl.BlockSpec((D, 2 * D), lambda b: (0, 0)),
            pl.BlockSpec((2 * D,), lambda b: (0,)),
        ],
        out_specs=pl.BlockSpec((B, D), lambda b: (0, 0)),
        out_shape=jax.ShapeDtypeStruct((B, D), jnp.float32),
        scratch_shapes=[
            pltpu.VMEM((N, D), jnp.bfloat16),
            pltpu.VMEM((N, D), jnp.bfloat16),
            pltpu.VMEM((N, D), jnp.float32),
            pltpu.VMEM((N, 1), jnp.float32),
        ],
    )(feats, Wq, bq, Wkv, bkv)


# speedup vs baseline: 33.3542x; 2.0210x over previous
import jax
import jax.numpy as jnp
from jax import lax
from jax.experimental import pallas as pl
from jax.experimental.pallas import tpu as pltpu

N = 4096
D = 128
KNN = 8
RT = 256
NT = N // RT
NEG = -3.0e38
EPS = 1e-12


def _agg_kernel(feats_ref, wq_ref, bq_ref, wkv_ref, bkv_ref, out_ref,
                nq_ref, nk_ref, v_ref, qn_ref):
    f = feats_ref[0].astype(jnp.bfloat16)
    q = jnp.dot(f, wq_ref[...].astype(jnp.bfloat16),
                preferred_element_type=jnp.float32) + bq_ref[...]
    kv = jnp.dot(f, wkv_ref[...].astype(jnp.bfloat16),
                 preferred_element_type=jnp.float32) + bkv_ref[...]
    km = kv[:, :D]
    v_ref[...] = kv[:, D:]

    qn = jnp.maximum(jnp.sqrt(jnp.sum(q * q, axis=1, keepdims=True)), EPS)
    kn = jnp.maximum(jnp.sqrt(jnp.sum(km * km, axis=1, keepdims=True)), EPS)
    nq_ref[...] = (q / qn).astype(jnp.bfloat16)
    nk_ref[...] = (km / kn).astype(jnp.bfloat16)
    qn_ref[...] = qn

    kn2_row = lax.dot_general(jnp.ones((1, D), jnp.float32), km * km,
                              (((1,), (1,)), ((), ())),
                              preferred_element_type=jnp.float32,
                              precision=lax.Precision.HIGHEST)
    kn_row = jnp.maximum(jnp.sqrt(kn2_row), EPS)

    ones_row = jnp.ones((1, RT), jnp.float32)
    inv_sqrt_d = 1.0 / (D ** 0.5)

    def tile_body(i, w):
        nqt = nq_ref[pl.ds(i * RT, RT), :]
        sim = lax.dot_general(nqt, nk_ref[...], (((1,), (1,)), ((), ())),
                              preferred_element_type=jnp.float32)
        colj = lax.broadcasted_iota(jnp.int32, (RT, N), 1)
        rowi = lax.broadcasted_iota(jnp.int32, (RT, N), 0) + i * RT
        diagm = colj == rowi
        r = jnp.where(diagm, NEG, sim)

        for _ in range(KNN):
            m = jnp.max(r, axis=1, keepdims=True)
            r = jnp.where(r == m, NEG, r)

        selmask = jnp.logical_and(r == NEG, jnp.logical_not(diagm))
        g = sim * ((qn_ref[pl.ds(i * RT, RT), :] * inv_sqrt_d) * kn_row)
        lmax = jnp.max(jnp.where(selmask, g, NEG), axis=1, keepdims=True)
        expg = jnp.where(selmask, jnp.exp(g - lmax), 0.0)
        zr = jnp.sum(expg, axis=1, keepdims=True)
        en = expg * (1.0 / zr)
        winc = lax.dot_general(ones_row, en, (((1,), (0,)), ((), ())),
                               preferred_element_type=jnp.float32,
                               precision=lax.Precision.HIGHEST)
        return w + winc

    w = lax.fori_loop(0, NT, tile_body, jnp.zeros((1, N), jnp.float32))
    z = lax.dot_general(w, v_ref[...], (((1,), (0,)), ((), ())),
                        preferred_element_type=jnp.float32,
                        precision=lax.Precision.HIGHEST)
    b = pl.program_id(0)
    out_ref[pl.ds(b, 1), :] = z * (1.0 / N)


@jax.jit
def kernel(feats, Wq, bq, Wkv, bkv):
    if feats.ndim == 2:
        feats = feats[None]
    B = feats.shape[0]
    return pl.pallas_call(
        _agg_kernel,
        grid=(B,),
        in_specs=[
            pl.BlockSpec((1, N, D), lambda b: (b, 0, 0)),
            pl.BlockSpec((D, D), lambda b: (0, 0)),
            pl.BlockSpec((D,), lambda b: (0,)),
            pl.BlockSpec((D, 2 * D), lambda b: (0, 0)),
            pl.BlockSpec((2 * D,), lambda b: (0,)),
        ],
        out_specs=pl.BlockSpec((B, D), lambda b: (0, 0)),
        out_shape=jax.ShapeDtypeStruct((B, D), jnp.float32),
        scratch_shapes=[
            pltpu.VMEM((N, D), jnp.bfloat16),
            pltpu.VMEM((N, D), jnp.bfloat16),
            pltpu.VMEM((N, D), jnp.float32),
            pltpu.VMEM((N, 1), jnp.float32),
        ],
    )(feats, Wq, bq, Wkv, bkv)


# speedup vs baseline: 39.7000x; 1.1903x over previous
import jax
import jax.numpy as jnp
from jax import lax
from jax.experimental import pallas as pl
from jax.experimental.pallas import tpu as pltpu

N = 4096
D = 128
KNN = 8
RT = 512
NT = N // RT
NEG = -3.0e38
NEG2 = -2.0e38
EPS = 1e-12


def _agg_kernel(feats_ref, wq_ref, bq_ref, wkv_ref, bkv_ref, out_ref,
                nq_ref, nk_ref, v_ref, qn_ref):
    f = feats_ref[0].astype(jnp.bfloat16)
    q = jnp.dot(f, wq_ref[...].astype(jnp.bfloat16),
                preferred_element_type=jnp.float32) + bq_ref[...]
    kv = jnp.dot(f, wkv_ref[...].astype(jnp.bfloat16),
                 preferred_element_type=jnp.float32) + bkv_ref[...]
    km = kv[:, :D]
    v_ref[...] = kv[:, D:]

    qn = jnp.maximum(jnp.sqrt(jnp.sum(q * q, axis=1, keepdims=True)), EPS)
    kn = jnp.maximum(jnp.sqrt(jnp.sum(km * km, axis=1, keepdims=True)), EPS)
    nq_ref[...] = (q / qn).astype(jnp.bfloat16)
    nk_ref[...] = (km / kn).astype(jnp.bfloat16)
    qn_ref[...] = qn

    kn2_row = lax.dot_general(jnp.ones((1, D), jnp.float32), km * km,
                              (((1,), (1,)), ((), ())),
                              preferred_element_type=jnp.float32,
                              precision=lax.Precision.HIGHEST)
    ka_row = jnp.maximum(jnp.sqrt(kn2_row), EPS) * (1.0 / (D ** 0.5))

    eye_rt = jnp.where(
        lax.broadcasted_iota(jnp.int32, (RT, RT), 0)
        == lax.broadcasted_iota(jnp.int32, (RT, RT), 1),
        1.0, 0.0).astype(jnp.float32)

    def tile_body(i, w):
        nqt = nq_ref[pl.ds(i * RT, RT), :]
        sim = lax.dot_general(nqt, nk_ref[...], (((1,), (1,)), ((), ())),
                              preferred_element_type=jnp.float32)
        t1 = sim * qn_ref[pl.ds(i * RT, RT), :]
        colj = lax.broadcasted_iota(jnp.int32, (RT, N), 1)
        rowi = lax.broadcasted_iota(jnp.int32, (RT, N), 0) + i * RT
        r = jnp.where(colj == rowi, NEG2, t1)

        for _ in range(KNN):
            m = jnp.max(r, axis=1, keepdims=True)
            r = jnp.where(r == m, NEG, r)

        selmask = r == NEG
        g = t1 * ka_row
        expg = jnp.where(selmask, jnp.exp(g), 0.0)
        inv_zr = 1.0 / jnp.sum(expg, axis=1, keepdims=True)
        inv_zr_t = lax.dot_general(inv_zr, eye_rt, (((0,), (0,)), ((), ())),
                                   preferred_element_type=jnp.float32,
                                   precision=lax.Precision.HIGHEST)
        winc = lax.dot_general(inv_zr_t, expg, (((1,), (0,)), ((), ())),
                               preferred_element_type=jnp.float32,
                               precision=lax.Precision.HIGHEST)
        return w + winc

    w = lax.fori_loop(0, NT, tile_body, jnp.zeros((1, N), jnp.float32))
    z = lax.dot_general(w, v_ref[...], (((1,), (0,)), ((), ())),
                        preferred_element_type=jnp.float32,
                        precision=lax.Precision.HIGHEST)
    b = pl.program_id(0)
    out_ref[pl.ds(b, 1), :] = z * (1.0 / N)


@jax.jit
def kernel(feats, Wq, bq, Wkv, bkv):
    if feats.ndim == 2:
        feats = feats[None]
    B = feats.shape[0]
    return pl.pallas_call(
        _agg_kernel,
        grid=(B,),
        in_specs=[
            pl.BlockSpec((1, N, D), lambda b: (b, 0, 0)),
            pl.BlockSpec((D, D), lambda b: (0, 0)),
            pl.BlockSpec((D,), lambda b: (0,)),
            pl.BlockSpec((D, 2 * D), lambda b: (0, 0)),
            pl.BlockSpec((2 * D,), lambda b: (0,)),
        ],
        out_specs=pl.BlockSpec((B, D), lambda b: (0, 0)),
        out_shape=jax.ShapeDtypeStruct((B, D), jnp.float32),
        scratch_shapes=[
            pltpu.VMEM((N, D), jnp.bfloat16),
            pltpu.VMEM((N, D), jnp.bfloat16),
            pltpu.VMEM((N, D), jnp.float32),
            pltpu.VMEM((N, 1), jnp.float32),
        ],
    )(feats, Wq, bq, Wkv, bkv)
